# baseline (speedup 1.0000x reference)
import jax
import jax.numpy as jnp
from jax.experimental import pallas as pl
from jax.experimental.pallas import tpu as pltpu

N = 4096
T = 4
D = 128
H = 64
C = 16

BLK1 = 256
BLKH = 512
BLK2 = 1024
NB1 = N // BLK1
NF = T * NB1
NBH = N // BLKH
NB2 = N // BLK2


def _fused_kernel(c_ref, x_ref, w1_ref, b1_ref, w2_ref, b2_ref, adj_ref,
                  out_ref, a_ref, h_ref, xw1_ref, hw2_ref):
    i = pl.program_id(0)

    @pl.when(i == 0)
    def _():
        xw1_ref[...] = jnp.dot(x_ref[...], w1_ref[...],
                               preferred_element_type=jnp.float32
                               ).astype(jnp.bfloat16)

    @pl.when(i < NB1)
    def _():
        a_ref[pl.ds(i * BLK1, BLK1), :] = (
            c_ref[0] * adj_ref[...]).astype(jnp.bfloat16)

    @pl.when(jnp.logical_and(i >= NB1, i < NF))
    def _():
        t = i // NB1
        r = (i % NB1) * BLK1
        rows = pl.ds(r, BLK1)
        a_ref[rows, :] = (
            a_ref[rows, :].astype(jnp.float32) + c_ref[t] * adj_ref[...]
        ).astype(jnp.bfloat16)

    @pl.when(jnp.logical_and(i >= NF, i < NF + NBH))
    def _():
        k = i - NF
        h_ref[pl.ds(k * BLKH, BLKH), :] = jax.nn.relu(
            jnp.dot(a_ref[pl.ds(k * BLKH, BLKH), :], xw1_ref[...],
                    preferred_element_type=jnp.float32) + b1_ref[...]
        ).astype(jnp.bfloat16)

    @pl.when(i == NF + NBH)
    def _():
        hw2_ref[...] = jnp.dot(h_ref[...], w2_ref[...],
                               preferred_element_type=jnp.float32
                               ).astype(jnp.bfloat16)

    @pl.when(i >= NF + NBH)
    def _():
        j = i - NF - NBH
        logits = jnp.dot(a_ref[pl.ds(j * BLK2, BLK2), :], hw2_ref[...],
                         preferred_element_type=jnp.float32) + b2_ref[...]
        m = jnp.max(logits, axis=-1, keepdims=True)
        e = jnp.exp(logits - m)
        out_ref[...] = e / jnp.sum(e, axis=-1, keepdims=True)


@jax.jit
def kernel(feats, adj, lam, W1, b1, W2, b2):
    x = feats[:, -1, :]
    one_m = 1.0 - lam
    c = jnp.stack([one_m ** (T - 1)]
                  + [lam * one_m ** (T - 1 - t) for t in range(1, T)])
    c = c.astype(jnp.float32)
    adj_flat = adj.reshape(T * N, N)

    out = pl.pallas_call(
        _fused_kernel,
        grid=(NF + NBH + NB2,),
        in_specs=[
            pl.BlockSpec(memory_space=pltpu.SMEM),
            pl.BlockSpec((N, D), lambda i: (0, 0)),
            pl.BlockSpec((D, H), lambda i: (0, 0)),
            pl.BlockSpec((1, H), lambda i: (0, 0)),
            pl.BlockSpec((H, C), lambda i: (0, 0)),
            pl.BlockSpec((1, C), lambda i: (0, 0)),
            pl.BlockSpec((BLK1, N),
                         lambda i: (jnp.minimum(i, NF - 1), 0)),
        ],
        out_specs=pl.BlockSpec(
            (BLK2, C), lambda i: (jnp.maximum(i - NF - NBH, 0), 0)),
        out_shape=jax.ShapeDtypeStruct((N, C), jnp.float32),
        scratch_shapes=[
            pltpu.VMEM((N, N), jnp.bfloat16),
            pltpu.VMEM((N, H), jnp.bfloat16),
            pltpu.VMEM((N, H), jnp.bfloat16),
            pltpu.VMEM((N, C), jnp.bfloat16),
        ],
    )(c, x, W1, b1.reshape(1, H), W2, b2.reshape(1, C), adj_flat)

    return out


# speedup vs baseline: 1.2205x; 1.2205x over previous
import jax
import jax.numpy as jnp
from jax.experimental import pallas as pl
from jax.experimental.pallas import tpu as pltpu

N = 4096
T = 4
D = 128
H = 64
C = 16

BLK1 = 128
BLK2 = 1024
NB1 = N // BLK1
NB2 = N // BLK2


def _fused_kernel(c_ref, x_ref, w1_ref, b1_ref, w2_ref, b2_ref, adj_ref,
                  out_ref, a_ref, h_ref, xw1_ref, hw2_ref):
    i = pl.program_id(0)

    @pl.when(i == 0)
    def _():
        xw1_ref[...] = jnp.dot(x_ref[...], w1_ref[...],
                               preferred_element_type=jnp.float32
                               ).astype(jnp.bfloat16)

    @pl.when(i < NB1)
    def _():
        a_blk = (c_ref[0] * adj_ref[0] + c_ref[1] * adj_ref[1]
                 + c_ref[2] * adj_ref[2] + c_ref[3] * adj_ref[3])
        a_bf = a_blk.astype(jnp.bfloat16)
        a_ref[pl.ds(i * BLK1, BLK1), :] = a_bf
        h_ref[pl.ds(i * BLK1, BLK1), :] = jax.nn.relu(
            jnp.dot(a_bf, xw1_ref[...],
                    preferred_element_type=jnp.float32) + b1_ref[...]
        ).astype(jnp.bfloat16)

    @pl.when(i == NB1)
    def _():
        hw2_ref[...] = jnp.dot(h_ref[...], w2_ref[...],
                               preferred_element_type=jnp.float32
                               ).astype(jnp.bfloat16)

    @pl.when(i >= NB1)
    def _():
        j = i - NB1
        logits = jnp.dot(a_ref[pl.ds(j * BLK2, BLK2), :], hw2_ref[...],
                         preferred_element_type=jnp.float32) + b2_ref[...]
        m = jnp.max(logits, axis=-1, keepdims=True)
        e = jnp.exp(logits - m)
        out_ref[...] = e / jnp.sum(e, axis=-1, keepdims=True)


@jax.jit
def kernel(feats, adj, lam, W1, b1, W2, b2):
    x = feats[:, -1, :]
    one_m = 1.0 - lam
    c = jnp.stack([one_m ** (T - 1)]
                  + [lam * one_m ** (T - 1 - t) for t in range(1, T)])
    c = c.astype(jnp.float32)

    out = pl.pallas_call(
        _fused_kernel,
        grid=(NB1 + NB2,),
        in_specs=[
            pl.BlockSpec(memory_space=pltpu.SMEM),
            pl.BlockSpec((N, D), lambda i: (0, 0)),
            pl.BlockSpec((D, H), lambda i: (0, 0)),
            pl.BlockSpec((1, H), lambda i: (0, 0)),
            pl.BlockSpec((H, C), lambda i: (0, 0)),
            pl.BlockSpec((1, C), lambda i: (0, 0)),
            pl.BlockSpec((T, BLK1, N),
                         lambda i: (0, jnp.minimum(i, NB1 - 1), 0)),
        ],
        out_specs=pl.BlockSpec((BLK2, C),
                               lambda i: (jnp.maximum(i - NB1, 0), 0)),
        out_shape=jax.ShapeDtypeStruct((N, C), jnp.float32),
        scratch_shapes=[
            pltpu.VMEM((N, N), jnp.bfloat16),
            pltpu.VMEM((N, H), jnp.bfloat16),
            pltpu.VMEM((N, H), jnp.bfloat16),
            pltpu.VMEM((N, C), jnp.bfloat16),
        ],
    )(c, x, W1, b1.reshape(1, H), W2, b2.reshape(1, C), adj)

    return out
